# baseline (speedup 1.0000x reference)
import functools

import jax
import jax.numpy as jnp
from jax import lax
from jax.experimental import pallas as pl
from jax.experimental.pallas import tpu as pltpu
from jax.experimental.pallas import tpu_sc as plsc

NC, NS, LANES = 2, 16, 16
NW = NC * NS


def _body(B, L, D, K, rows_per_w, CH, hs_hbm, perm_hbm, shifts_hbm, out_hbm,
          perm_v, shifts_v, bufs, idx_sem, gsems, wsems):
    wid = lax.axis_index("s") * NC + lax.axis_index("c")
    wpb = NW // B
    b = wid // wpb
    r0 = K + (wid % wpb) * rows_per_w
    base_row = b * L + r0

    pltpu.async_copy(perm_hbm, perm_v, idx_sem)
    pltpu.async_copy(shifts_hbm, shifts_v, idx_sem)
    pltpu.make_async_copy(perm_hbm, perm_v, idx_sem).wait()
    pltpu.make_async_copy(shifts_hbm, shifts_v, idx_sem).wait()
    shift = plsc.load_gather(shifts_v, [jnp.full((LANES,), b, jnp.int32)])
    row_base = b * L

    nchunks = rows_per_w // CH
    nbuf = len(bufs)
    lanes = lax.iota(jnp.int32, LANES)

    def fire_gather(c, slot):
        pos = (lanes + shift + (r0 + c * CH)) & (L - 1)
        gidx = plsc.load_gather(perm_v, [pos]) + row_base
        pltpu.async_copy(hs_hbm.at[gidx], bufs[slot], gsems[slot])

    def wait_gather(slot):
        pltpu.make_async_copy(hs_hbm.at[pl.ds(0, CH)], bufs[slot],
                              gsems[slot]).wait()

    def fire_write(c, slot):
        pltpu.async_copy(
            bufs[slot], out_hbm.at[pl.ds(base_row + c * CH, CH)], wsems[slot])

    def wait_write(slot):
        pltpu.make_async_copy(bufs[slot], out_hbm.at[pl.ds(base_row, CH)],
                              wsems[slot]).wait()

    for s in range(min(nbuf, nchunks)):
        fire_gather(s, s)

    ncycles = -(-nchunks // nbuf)

    def ring_body(i, _):
        for s in range(nbuf):
            c = i * nbuf + s

            @pl.when(c < nchunks)
            def _():
                wait_gather(s)
                fire_write(c, s)

            @pl.when(c + nbuf < nchunks)
            def _():
                wait_write(s)
                fire_gather(c + nbuf, s)
        return ()

    lax.fori_loop(0, ncycles, ring_body, ())
    for s in range(min(nbuf, nchunks)):
        wait_write(s)


def _tc_body(perm_ref, shifts_ref, hs_blk, out_blk):
    del perm_ref, shifts_ref
    out_blk[...] = hs_blk[...]


def _tc_gather(hs, perm, shifts_i, K):
    B, L, D = hs.shape
    hs4 = hs.reshape(B, L, 8, D // 8)
    grid = (B, K)

    def in_map(b, i, perm_ref, shifts_ref):
        return (b, perm_ref[(i + shifts_ref[b]) & (L - 1)], 0, 0)

    out4 = pl.pallas_call(
        _tc_body,
        grid_spec=pltpu.PrefetchScalarGridSpec(
            num_scalar_prefetch=2,
            grid=grid,
            in_specs=[pl.BlockSpec((1, 1, 8, D // 8), in_map)],
            out_specs=pl.BlockSpec((1, 1, 8, D // 8),
                                   lambda b, i, p, s: (b, i, 0, 0)),
        ),
        out_shape=jax.ShapeDtypeStruct((B, K, 8, D // 8), jnp.float32),
    )(perm, shifts_i, hs4)
    return out4.reshape(B, K, D)


def kernel(hidden_states, base_perm, shifts):
    B, L, D = hidden_states.shape
    K = 512
    assert (B * (L - K)) % NW == 0 and L & (L - 1) == 0
    rows_per_w = (B * (L - K)) // NW
    CH = LANES
    NBUF = 6

    hs2 = hidden_states.reshape(B * L, D)
    perm = base_perm.astype(jnp.int32)
    shifts_i = shifts.astype(jnp.int32)

    mesh = plsc.VectorSubcoreMesh(core_axis_name="c", subcore_axis_name="s")
    run = pl.kernel(
        functools.partial(_body, B, L, D, K, rows_per_w, CH),
        out_type=jax.ShapeDtypeStruct((B * L, D), jnp.float32),
        mesh=mesh,
        scratch_types=[
            pltpu.VMEM((L,), jnp.int32),
            pltpu.VMEM((B,), jnp.int32),
            [pltpu.VMEM((CH, D), jnp.float32) for _ in range(NBUF)],
            pltpu.SemaphoreType.DMA,
            [pltpu.SemaphoreType.DMA for _ in range(NBUF)],
            [pltpu.SemaphoreType.DMA for _ in range(NBUF)],
        ],
        compiler_params=pltpu.CompilerParams(needs_layout_passes=False),
    )
    sc_out = run(hs2, perm, shifts_i).reshape(B, L, D)
    tc_out = _tc_gather(hidden_states, perm, shifts_i, K)
    return lax.dynamic_update_slice(sc_out, tc_out, (0, 0, 0))


# speedup vs baseline: 9.3686x; 9.3686x over previous
import functools

import jax
import jax.numpy as jnp
from jax import lax
from jax.experimental import pallas as pl
from jax.experimental.pallas import tpu as pltpu
from jax.experimental.pallas import tpu_sc as plsc

NC, NS, LANES = 2, 16, 16
NW = NC * NS


def _body(B, L, D, rows_per_w, CH, hs_hbm, perm_hbm, shifts_hbm, out_hbm,
          perm_v, shifts_v, bufs, idx_sem, gsems, wsems):
    wid = lax.axis_index("s") * NC + lax.axis_index("c")
    base_row = wid * rows_per_w
    b = base_row // L
    r0 = base_row % L

    pltpu.async_copy(perm_hbm, perm_v, idx_sem)
    pltpu.async_copy(shifts_hbm, shifts_v, idx_sem)
    pltpu.make_async_copy(perm_hbm, perm_v, idx_sem).wait()
    pltpu.make_async_copy(shifts_hbm, shifts_v, idx_sem).wait()
    shift = plsc.load_gather(shifts_v, [jnp.full((LANES,), b, jnp.int32)])
    row_base = b * L

    nchunks = rows_per_w // CH
    nbuf = len(bufs)
    lanes = lax.iota(jnp.int32, LANES)

    def fire_gather(c, slot):
        pos = (lanes + shift + (r0 + c * CH)) & (L - 1)
        gidx = plsc.load_gather(perm_v, [pos]) + row_base
        pltpu.async_copy(hs_hbm.at[gidx], bufs[slot], gsems[slot])

    def wait_gather(slot):
        pltpu.make_async_copy(hs_hbm.at[pl.ds(0, CH)], bufs[slot],
                              gsems[slot]).wait()

    def fire_write(c, slot):
        pltpu.async_copy(
            bufs[slot], out_hbm.at[pl.ds(base_row + c * CH, CH)], wsems[slot])

    def wait_write(slot):
        pltpu.make_async_copy(bufs[slot], out_hbm.at[pl.ds(base_row, CH)],
                              wsems[slot]).wait()

    for s in range(min(nbuf, nchunks)):
        fire_gather(s, s)

    ncycles = -(-nchunks // nbuf)

    def ring_body(i, _):
        for s in range(nbuf):
            c = i * nbuf + s

            @pl.when(c < nchunks)
            def _():
                wait_gather(s)
                fire_write(c, s)

            @pl.when(c + nbuf < nchunks)
            def _():
                wait_write(s)
                fire_gather(c + nbuf, s)
        return ()

    lax.fori_loop(0, ncycles, ring_body, ())
    for s in range(min(nbuf, nchunks)):
        wait_write(s)


def kernel(hidden_states, base_perm, shifts):
    B, L, D = hidden_states.shape
    assert (B * L) % NW == 0 and L & (L - 1) == 0
    rows_per_w = (B * L) // NW
    CH = LANES
    NBUF = 6

    hs2 = hidden_states.reshape(B * L, D)
    perm = base_perm.astype(jnp.int32)
    shifts_i = shifts.astype(jnp.int32)

    mesh = plsc.VectorSubcoreMesh(core_axis_name="c", subcore_axis_name="s")
    run = pl.kernel(
        functools.partial(_body, B, L, D, rows_per_w, CH),
        out_type=jax.ShapeDtypeStruct((B * L, D), jnp.float32),
        mesh=mesh,
        scratch_types=[
            pltpu.VMEM((L,), jnp.int32),
            pltpu.VMEM((B,), jnp.int32),
            [pltpu.VMEM((CH, D), jnp.float32) for _ in range(NBUF)],
            pltpu.SemaphoreType.DMA,
            [pltpu.SemaphoreType.DMA for _ in range(NBUF)],
            [pltpu.SemaphoreType.DMA for _ in range(NBUF)],
        ],
        compiler_params=pltpu.CompilerParams(needs_layout_passes=False),
    )
    out = run(hs2, perm, shifts_i)
    return out.reshape(B, L, D)


# speedup vs baseline: 9.4149x; 1.0049x over previous
import functools

import jax
import jax.numpy as jnp
from jax import lax
from jax.experimental import pallas as pl
from jax.experimental.pallas import tpu as pltpu
from jax.experimental.pallas import tpu_sc as plsc

NC, NS, LANES = 2, 16, 16
NW = NC * NS


def _body(B, L, D, rows_per_w, CH, hs_hbm, perm_hbm, shifts_hbm, out_hbm,
          perm_v, shifts_v, bufs, idx_sem, gsems, wsems):
    wid = lax.axis_index("s") * NC + lax.axis_index("c")
    base_row = wid * rows_per_w
    b = base_row // L
    r0 = base_row % L

    pltpu.async_copy(perm_hbm, perm_v, idx_sem)
    pltpu.async_copy(shifts_hbm, shifts_v, idx_sem)
    pltpu.make_async_copy(perm_hbm, perm_v, idx_sem).wait()
    pltpu.make_async_copy(shifts_hbm, shifts_v, idx_sem).wait()
    shift = plsc.load_gather(shifts_v, [jnp.full((LANES,), b, jnp.int32)])
    row_base = b * L

    nchunks = rows_per_w // CH
    nbuf = len(bufs)
    lanes = lax.iota(jnp.int32, LANES)

    def fire_gather(c, slot):
        pos = (lanes + shift + (r0 + c * CH)) & (L - 1)
        gidx = plsc.load_gather(perm_v, [pos]) + row_base
        pltpu.async_copy(hs_hbm.at[gidx], bufs[slot], gsems[slot])

    def wait_gather(slot):
        pltpu.make_async_copy(hs_hbm.at[pl.ds(0, CH)], bufs[slot],
                              gsems[slot]).wait()

    def fire_write(c, slot):
        pltpu.async_copy(
            bufs[slot], out_hbm.at[pl.ds(base_row + c * CH, CH)], wsems[slot])

    def wait_write(slot):
        pltpu.make_async_copy(bufs[slot], out_hbm.at[pl.ds(base_row, CH)],
                              wsems[slot]).wait()

    for s in range(min(nbuf, nchunks)):
        fire_gather(s, s)

    ncycles = -(-nchunks // nbuf)

    def ring_body(i, _):
        for s in range(nbuf):
            c = i * nbuf + s

            @pl.when(c < nchunks)
            def _():
                wait_gather(s)
                fire_write(c, s)

            @pl.when(c + nbuf < nchunks)
            def _():
                wait_write(s)
                fire_gather(c + nbuf, s)
        return ()

    lax.fori_loop(0, ncycles, ring_body, ())
    for s in range(min(nbuf, nchunks)):
        wait_write(s)


def kernel(hidden_states, base_perm, shifts):
    B, L, D = hidden_states.shape
    assert (B * L) % NW == 0 and L & (L - 1) == 0
    rows_per_w = (B * L) // NW
    CH = LANES
    NBUF = 7

    hs2 = hidden_states.reshape(B * L, D)
    perm = base_perm.astype(jnp.int32)
    shifts_i = shifts.astype(jnp.int32)

    mesh = plsc.VectorSubcoreMesh(core_axis_name="c", subcore_axis_name="s")
    run = pl.kernel(
        functools.partial(_body, B, L, D, rows_per_w, CH),
        out_type=jax.ShapeDtypeStruct((B * L, D), jnp.float32),
        mesh=mesh,
        scratch_types=[
            pltpu.VMEM((L,), jnp.int32),
            pltpu.VMEM((B,), jnp.int32),
            [pltpu.VMEM((CH, D), jnp.float32) for _ in range(NBUF)],
            pltpu.SemaphoreType.DMA,
            [pltpu.SemaphoreType.DMA for _ in range(NBUF)],
            [pltpu.SemaphoreType.DMA for _ in range(NBUF)],
        ],
        compiler_params=pltpu.CompilerParams(needs_layout_passes=False),
    )
    out = run(hs2, perm, shifts_i)
    return out.reshape(B, L, D)
